# baseline (speedup 1.0000x reference)
import functools

import jax
import jax.numpy as jnp
from jax import lax
from jax.experimental import pallas as pl
from jax.experimental.pallas import tpu as pltpu
from jax.experimental.pallas import tpu_sc as plsc

EMBED_DIM = 64
N_ACTION = 2
N_CORES = 2
N_SUBCORES = 16
NW = N_CORES * N_SUBCORES


@functools.lru_cache(maxsize=None)
def _make_gather(batch: int):
    assert batch % (8 * NW) == 0
    b_per_w = batch // NW
    mesh = plsc.VectorSubcoreMesh(core_axis_name="c", subcore_axis_name="s")

    @functools.partial(
        pl.kernel,
        mesh=mesh,
        out_type=jax.ShapeDtypeStruct((batch, EMBED_DIM), jnp.float32),
        scratch_types=[
            pltpu.VMEM((b_per_w,), jnp.int32),
            pltpu.VMEM((b_per_w, EMBED_DIM), jnp.float32),
            pltpu.SemaphoreType.DMA,
        ],
        compiler_params=pltpu.CompilerParams(use_tc_tiling_on_sc=False),
    )
    def gather(idx_hbm, table_hbm, out_hbm, idx_v, rows_v, sem):
        wid = lax.axis_index("s") * N_CORES + lax.axis_index("c")
        base = wid * b_per_w
        pltpu.sync_copy(idx_hbm.at[pl.ds(base, b_per_w)], idx_v)
        pltpu.async_copy(table_hbm.at[idx_v], rows_v, sem).wait()
        pltpu.sync_copy(rows_v, out_hbm.at[pl.ds(base, b_per_w)])

    return gather


def _linear_body(x_ref, w_ref, b_ref, o_ref):
    o_ref[...] = lax.dot_general(
        x_ref[...], w_ref[...],
        (((1,), (1,)), ((), ())),
        preferred_element_type=jnp.float32,
    ) + b_ref[...]


@functools.lru_cache(maxsize=None)
def _make_linear(batch: int, blk: int = 2048):
    return pl.pallas_call(
        _linear_body,
        grid=(batch // blk,),
        in_specs=[
            pl.BlockSpec((blk, EMBED_DIM), lambda i: (i, 0)),
            pl.BlockSpec((N_ACTION, EMBED_DIM), lambda i: (0, 0)),
            pl.BlockSpec((1, N_ACTION), lambda i: (0, 0)),
        ],
        out_specs=pl.BlockSpec((blk, N_ACTION), lambda i: (i, 0)),
        out_shape=jax.ShapeDtypeStruct((batch, N_ACTION), jnp.float32),
    )


def kernel(user_idx, table, W, b):
    batch = user_idx.shape[0]
    x = _make_gather(batch)(user_idx.astype(jnp.int32), table)
    return _make_linear(batch)(x, W, b.reshape(1, N_ACTION))


# speedup vs baseline: 2.0956x; 2.0956x over previous
import functools

import jax
import jax.numpy as jnp
from jax import lax
from jax.experimental import pallas as pl
from jax.experimental.pallas import tpu as pltpu
from jax.experimental.pallas import tpu_sc as plsc

EMBED_DIM = 64
N_ACTION = 2
GRP = 8
N_CORES = 2
N_SUBCORES = 16
NW = N_CORES * N_SUBCORES
CHUNK = 32
LANES = 16


@functools.lru_cache(maxsize=None)
def _make_fused(batch: int, n_rows: int):
    assert batch % (LANES * NW) == 0
    b_per_w = batch // NW
    n_chunks = b_per_w // CHUNK
    mesh = plsc.VectorSubcoreMesh(core_axis_name="c", subcore_axis_name="s")

    @functools.partial(
        pl.kernel,
        mesh=mesh,
        out_type=jax.ShapeDtypeStruct((batch * N_ACTION,), jnp.float32),
        scratch_types=[
            pltpu.VMEM((b_per_w,), jnp.int32),
            pltpu.VMEM((b_per_w,), jnp.int32),
            pltpu.VMEM((CHUNK, GRP, EMBED_DIM), jnp.float32),
            pltpu.VMEM((256,), jnp.float32),
            pltpu.VMEM((b_per_w * N_ACTION,), jnp.float32),
            pltpu.SemaphoreType.DMA,
        ],
        compiler_params=pltpu.CompilerParams(
            use_tc_tiling_on_sc=True, needs_layout_passes=False
        ),
    )
    def fused(idx_hbm, table_hbm, wb_hbm, out_hbm,
              idx_v, row_v, rows_v, wb_v, out_v, sem):
        wid = lax.axis_index("s") * N_CORES + lax.axis_index("c")
        base = wid * b_per_w
        pltpu.sync_copy(idx_hbm.at[pl.ds(base, b_per_w)], idx_v)
        pltpu.sync_copy(wb_hbm, wb_v)

        def split_body(k, _):
            v = idx_v[pl.ds(k * LANES, LANES)]
            row_v[pl.ds(k * LANES, LANES)] = v & 7
            return 0

        lax.fori_loop(0, b_per_w // LANES, split_body, 0)

        w_vecs = [
            [wb_v[pl.ds(a * EMBED_DIM + k * LANES, LANES)] for k in range(4)]
            for a in range(N_ACTION)
        ]
        bias_vec = wb_v[pl.ds(2 * EMBED_DIM, LANES)]
        lane_iota = lax.iota(jnp.int32, LANES)

        def chunk_body(c, _):
            copies = []
            for jb in range(CHUNK // LANES):
                iv = idx_v[pl.ds(c * CHUNK + jb * LANES, LANES)]
                for j2 in range(LANES):
                    g = iv[j2] >> 3
                    copies.append(
                        pltpu.async_copy(
                            table_hbm.at[g], rows_v.at[jb * LANES + j2], sem
                        )
                    )
            for cp in copies:
                cp.wait()

            def blk_body(jb, _):
                slot_vec = jb * LANES + lane_iota
                r_vec = row_v[pl.ds(c * CHUNK + jb * LANES, LANES)]
                acc = [jnp.full((LANES,), bias_vec[a], jnp.float32)
                       for a in range(N_ACTION)]
                for d in range(EMBED_DIM):
                    d_vec = jnp.full((LANES,), d, jnp.int32)
                    val = plsc.load_gather(rows_v, [slot_vec, r_vec, d_vec])
                    for a in range(N_ACTION):
                        acc[a] = acc[a] + val * w_vecs[a][d // 16][d % 16]
                opos = (c * CHUNK + jb * LANES + lane_iota) * N_ACTION
                for a in range(N_ACTION):
                    plsc.store_scatter(out_v, [opos + a], acc[a])
                return 0

            lax.fori_loop(0, CHUNK // LANES, blk_body, 0)
            return 0

        lax.fori_loop(0, n_chunks, chunk_body, 0)
        pltpu.sync_copy(
            out_v, out_hbm.at[pl.ds(base * N_ACTION, b_per_w * N_ACTION)]
        )

    return fused


def kernel(user_idx, table, W, b):
    batch = user_idx.shape[0]
    n_rows = table.shape[0]
    table3 = table.reshape(n_rows // GRP, GRP, EMBED_DIM)
    wb = jnp.zeros((256,), jnp.float32)
    wb = wb.at[: N_ACTION * EMBED_DIM].set(W.reshape(-1))
    wb = wb.at[2 * EMBED_DIM : 2 * EMBED_DIM + N_ACTION].set(b)
    flat = _make_fused(batch, n_rows)(user_idx.astype(jnp.int32), table3, wb)
    return flat.reshape(batch, N_ACTION)


# speedup vs baseline: 3.7033x; 1.7671x over previous
import functools

import jax
import jax.numpy as jnp
from jax import lax
from jax.experimental import pallas as pl
from jax.experimental.pallas import tpu as pltpu
from jax.experimental.pallas import tpu_sc as plsc

EMBED_DIM = 64
N_ACTION = 2
N_CORES = 2
N_SUBCORES = 16
NW = N_CORES * N_SUBCORES
LANES = 16
ROW = 128
RBLK = 80
CHUNK = 256


def _proj_body(w_ref, b_ref, x_ref, o0_ref, o1_ref):
    y = lax.dot_general(
        w_ref[...], x_ref[...],
        (((1,), (0,)), ((), ())),
        preferred_element_type=jnp.float32,
        precision=lax.Precision.HIGHEST,
    ) + b_ref[...]
    o0_ref[...] = y[0:1, :].reshape(RBLK, ROW)
    o1_ref[...] = y[1:2, :].reshape(RBLK, ROW)


@functools.lru_cache(maxsize=None)
def _make_proj(n_rows: int):
    n_prows = (n_rows + ROW - 1) // ROW
    grid = (n_prows + RBLK - 1) // RBLK
    cblk = RBLK * ROW
    return pl.pallas_call(
        _proj_body,
        grid=(grid,),
        in_specs=[
            pl.BlockSpec((N_ACTION, EMBED_DIM), lambda i: (0, 0)),
            pl.BlockSpec((N_ACTION, 1), lambda i: (0, 0)),
            pl.BlockSpec((EMBED_DIM, cblk), lambda i: (0, i)),
        ],
        out_specs=[
            pl.BlockSpec((RBLK, ROW), lambda i: (i, 0)),
            pl.BlockSpec((RBLK, ROW), lambda i: (i, 0)),
        ],
        out_shape=[
            jax.ShapeDtypeStruct((n_prows, ROW), jnp.float32),
            jax.ShapeDtypeStruct((n_prows, ROW), jnp.float32),
        ],
    )


@functools.lru_cache(maxsize=None)
def _make_lookup(batch: int, n_prows: int):
    assert batch % (CHUNK * NW) == 0
    b_per_w = batch // NW
    n_chunks = b_per_w // CHUNK
    mesh = plsc.VectorSubcoreMesh(core_axis_name="c", subcore_axis_name="s")

    @functools.partial(
        pl.kernel,
        mesh=mesh,
        out_type=jax.ShapeDtypeStruct((N_ACTION, batch), jnp.float32),
        scratch_types=[
            pltpu.VMEM((b_per_w,), jnp.int32),
            pltpu.VMEM((b_per_w,), jnp.int32),
            pltpu.VMEM((CHUNK, ROW), jnp.float32),
            pltpu.VMEM((CHUNK, ROW), jnp.float32),
            pltpu.VMEM((b_per_w,), jnp.float32),
            pltpu.VMEM((b_per_w,), jnp.float32),
            pltpu.SemaphoreType.DMA,
        ],
        compiler_params=pltpu.CompilerParams(
            use_tc_tiling_on_sc=True, needs_layout_passes=False
        ),
    )
    def lookup(idx_hbm, y0_hbm, y1_hbm, out_hbm,
               idx_v, row_v, g0_v, g1_v, out0_v, out1_v, sem):
        wid = lax.axis_index("s") * N_CORES + lax.axis_index("c")
        base = wid * b_per_w
        pltpu.sync_copy(idx_hbm.at[pl.ds(base, b_per_w)], idx_v)

        def split_body(k, _):
            v = idx_v[pl.ds(k * LANES, LANES)]
            row_v[pl.ds(k * LANES, LANES)] = v >> 7
            return 0

        lax.fori_loop(0, b_per_w // LANES, split_body, 0)
        lane_iota = lax.iota(jnp.int32, LANES)

        def chunk_body(c, _):
            rows = row_v.at[pl.ds(c * CHUNK, CHUNK)]
            cp0 = pltpu.async_copy(y0_hbm.at[rows], g0_v, sem)
            cp1 = pltpu.async_copy(y1_hbm.at[rows], g1_v, sem)
            cp0.wait()
            cp1.wait()

            def blk_body(jb, _):
                iv = idx_v[pl.ds(c * CHUNK + jb * LANES, LANES)]
                slot = jb * LANES + lane_iota
                col = iv & (ROW - 1)
                v0 = plsc.load_gather(g0_v, [slot, col])
                v1 = plsc.load_gather(g1_v, [slot, col])
                out0_v[pl.ds(c * CHUNK + jb * LANES, LANES)] = v0
                out1_v[pl.ds(c * CHUNK + jb * LANES, LANES)] = v1
                return 0

            lax.fori_loop(0, CHUNK // LANES, blk_body, 0)
            return 0

        lax.fori_loop(0, n_chunks, chunk_body, 0)
        pltpu.sync_copy(out0_v, out_hbm.at[0, pl.ds(base, b_per_w)])
        pltpu.sync_copy(out1_v, out_hbm.at[1, pl.ds(base, b_per_w)])

    return lookup


def kernel(user_idx, table, W, b):
    batch = user_idx.shape[0]
    n_rows = table.shape[0]
    n_prows = (n_rows + ROW - 1) // ROW
    table_t = table.T
    y0, y1 = _make_proj(n_rows)(W, b.reshape(N_ACTION, 1), table_t)
    out_t = _make_lookup(batch, n_prows)(user_idx.astype(jnp.int32), y0, y1)
    return out_t.T
